# baseline (speedup 1.0000x reference)
import functools

import jax
import jax.numpy as jnp
from jax import lax
from jax.experimental import pallas as pl
from jax.experimental.pallas import tpu as pltpu
from jax.experimental.pallas import tpu_sc as plsc

N = 10000
E = 160000
T = 320000
H = 256
CENTERS = 80
TRIP = 40
NG = 64

_NC = 2
_NS = 16
_NW = _NC * _NS
_B = 40


def _silu(x):
    return x * jax.nn.sigmoid(x)


def _bn(x):
    m = jnp.mean(x, axis=0)
    v = jnp.var(x, axis=0)
    return (x - m) / jnp.sqrt(v + 1e-5)


def _rbf(d, vmin, vmax, bins):
    centers = jnp.linspace(vmin, vmax, bins)
    gamma = 1.0 / ((vmax - vmin) / (bins - 1))
    return jnp.exp(-gamma * (d - centers) ** 2)


def _mm_body(x_ref, w_ref, b_ref, o_ref):
    o_ref[...] = (
        jnp.dot(x_ref[...], w_ref[...], preferred_element_type=jnp.float32)
        + b_ref[...]
    )


def _mm(x, W, b, bm=1000):
    R, K = x.shape
    O = W.shape[1]
    return pl.pallas_call(
        _mm_body,
        grid=(R // bm,),
        in_specs=[
            pl.BlockSpec((bm, K), lambda r: (r, 0)),
            pl.BlockSpec((K, O), lambda r: (0, 0)),
            pl.BlockSpec((1, O), lambda r: (0, 0)),
        ],
        out_specs=pl.BlockSpec((bm, O), lambda r: (r, 0)),
        out_shape=jax.ShapeDtypeStruct((R, O), jnp.float32),
    )(x, W, b.reshape(1, -1))


_CT = 64


@functools.lru_cache(maxsize=None)
def _make_egc_sc(nt, nseg):
    Ct = _CT
    nchunk = nseg // Ct
    assert nchunk * Ct == nseg

    mesh = plsc.VectorSubcoreMesh(core_axis_name="c", subcore_axis_name="s")

    @functools.partial(
        pl.kernel,
        mesh=mesh,
        compiler_params=pltpu.CompilerParams(needs_layout_passes=False),
        out_type=[
            jax.ShapeDtypeStruct((nt + 8, H), jnp.float32),
            jax.ShapeDtypeStruct((nseg, H), jnp.float32),
            jax.ShapeDtypeStruct((nseg, H), jnp.float32),
        ],
        scratch_types=[
            pltpu.VMEM((nchunk + 17,), jnp.int32),
            pltpu.VMEM((64,), jnp.int32),
            pltpu.VMEM((64,), jnp.int32),
            pltpu.VMEM((64,), jnp.int32),
            pltpu.VMEM((64, H), jnp.float32),
            pltpu.VMEM((64, H), jnp.float32),
            pltpu.VMEM((64, H), jnp.float32),
            pltpu.VMEM((64, H), jnp.float32),
            pltpu.VMEM((Ct + 1, H), jnp.float32),
            pltpu.VMEM((Ct + 1, H), jnp.float32),
            pltpu.SemaphoreType.DMA,
        ],
    )
    def egc_sc(is_hbm, jp_hbm, rs_hbm, es_hbm, ed_hbm, bh_hbm, eg_hbm,
               yg_hbm, ssh_hbm, ss_hbm,
               rs_v, ivb, jvb, locw, esb, edb, bhb, egb, acc_m, acc_s, sem):
        w = lax.axis_index("s") * _NC + lax.axis_index("c")
        pltpu.sync_copy(rs_hbm, rs_v)

        def zrow(r, cr):
            for cc in range(H // 16):
                sl0 = pl.ds(cc * 16, 16)
                acc_m[r, sl0] = jnp.zeros((16,), jnp.float32)
                acc_s[r, sl0] = jnp.zeros((16,), jnp.float32)
            return cr

        lax.fori_loop(0, Ct + 1, zrow, 0, unroll=False)

        cntw = (nchunk - w + _NW - 1) // _NW
        iota16 = lax.broadcasted_iota(jnp.int32, (16,), 0)

        def chunk_body(k, carry):
            c = w + _NW * k
            seg_base = c * Ct
            bv = rs_v[pl.ds(c, 16)]
            start = bv[0]
            end = bv[1]
            ga = (start // 8) * 8
            ngr = jnp.maximum((end - ga + 63) // 64, 0)

            def gbody(g, cr2):
                bp = ga + g * 64
                pltpu.sync_copy(is_hbm.at[pl.ds(bp, 64)], ivb)
                pltpu.sync_copy(jp_hbm.at[pl.ds(bp, 64)], jvb)
                for q in range(4):
                    sl = pl.ds(q * 16, 16)
                    pvec = iota16 + (bp + q * 16)
                    valid = (pvec >= start) & (pvec < end)
                    locw[sl] = jnp.where(valid, pvec, nt)
                c1 = pltpu.async_copy(es_hbm.at[ivb], esb, sem)
                c2 = pltpu.async_copy(ed_hbm.at[jvb], edb, sem)
                c3 = pltpu.async_copy(bh_hbm.at[jvb], bhb, sem)
                c4 = pltpu.async_copy(eg_hbm.at[pl.ds(bp, 64)], egb, sem)
                c1.wait()
                c2.wait()
                c3.wait()
                c4.wait()

                def grow(r, cr3):
                    for cc in range(H // 16):
                        slc = pl.ds(cc * 16, 16)
                        yg = esb[r, slc] + edb[r, slc] + egb[r, slc]
                        sig = 1.0 / (1.0 + jnp.exp(-yg))
                        m = bhb[r, slc] * sig
                        esb[r, slc] = yg
                        edb[r, slc] = sig
                        bhb[r, slc] = m
                    return cr3

                lax.fori_loop(0, 64, grow, 0, unroll=False)
                c5 = pltpu.async_copy(esb, yg_hbm.at[locw], sem)
                for q in range(4):
                    sl = pl.ds(q * 16, 16)
                    iv = ivb[sl]
                    pvec = iota16 + (bp + q * 16)
                    valid = (pvec >= start) & (pvec < end)
                    lv = jnp.where(valid, iv - seg_base, Ct)
                    for rr in range(16):
                        rowi = jnp.zeros((16,), jnp.int32) + lv[rr]
                        for cc in range(H // 16):
                            slc = pl.ds(cc * 16, 16)
                            ci = iota16 + cc * 16
                            plsc.addupdate_scatter(
                                acc_m, [rowi, ci], bhb[q * 16 + rr, slc])
                            plsc.addupdate_scatter(
                                acc_s, [rowi, ci], edb[q * 16 + rr, slc])
                c5.wait()
                return cr2

            lax.fori_loop(0, ngr, gbody, 0, unroll=False)
            pltpu.sync_copy(acc_m.at[pl.ds(0, Ct)], ssh_hbm.at[pl.ds(seg_base, Ct)])
            pltpu.sync_copy(acc_s.at[pl.ds(0, Ct)], ss_hbm.at[pl.ds(seg_base, Ct)])
            lax.fori_loop(0, Ct, zrow, 0, unroll=False)
            return carry

        lax.fori_loop(0, cntw, chunk_body, 0, unroll=False)

    return egc_sc


_N_PAD = 10240


def _prep(seg_ids, other_ids, nseg_pad):
    perm = jnp.argsort(seg_ids).astype(jnp.int32)
    i_s = seg_ids[perm].astype(jnp.int32)
    jp = other_ids[perm].astype(jnp.int32)
    nchunk = nseg_pad // _CT
    bounds = (jnp.arange(nchunk + 1, dtype=jnp.int32) * _CT)
    rs = jnp.searchsorted(i_s, bounds).astype(jnp.int32)
    rs = jnp.concatenate([rs, jnp.zeros((16,), jnp.int32)])
    pad = jnp.zeros((128,), jnp.int32)
    return perm, jnp.concatenate([i_s, pad]), jnp.concatenate([jp, pad]), rs


def _egc(node, edge_s, i_s, jp, rs, p, nseg_pad, n_seg, egc_sc):
    es = _mm(node, p['sgW'], p['sgb'])
    ed = _mm(node, p['dgW'], p['dgb'])
    bh = _mm(node, p['duW'], p['dub'])
    su = _mm(node, p['suW'], p['sub'])
    eg = _mm(edge_s, p['egW'], p['egb'])
    nt = edge_s.shape[0]
    yg, ssh, ss = egc_sc(i_s, jp, rs, es, ed, bh, eg)
    yg = yg[:nt]
    h = ssh[:n_seg] / (ss[:n_seg] + 1e-6)
    xq = _silu(_bn(su + h))
    yq = _silu(_bn(yg))
    return node + xq, edge_s + yq


def kernel(x, edge_index, edge_index_triplets, dist, angle, batch, params):
    ie = edge_index[0]
    je = edge_index[1]
    it = edge_index_triplets[0]
    jt = edge_index_triplets[1]

    perm_e, ie_s, je_p, rs_e = _prep(ie, je, _N_PAD)
    rank_e = jnp.zeros((E,), jnp.int32).at[perm_e].set(
        jnp.arange(E, dtype=jnp.int32), unique_indices=True)
    itp = rank_e[it]
    jtp = rank_e[jt]
    perm_t, i_s_t, jp_t, rs_t = _prep(itp, jtp, E)

    xh = _silu(_bn(x @ params['atom']['W'] + params['atom']['b']))
    y = _rbf(dist[perm_e], 0.0, 8.0, CENTERS)
    y = _silu(_bn(_mm(y, params['edge1']['W'], params['edge1']['b'])))
    y = _silu(_bn(_mm(y, params['edge2']['W'], params['edge2']['b'])))
    z = _rbf(angle[perm_t], -1.0, 1.0, TRIP)
    z = _silu(_bn(_mm(z, params['ang1']['W'], params['ang1']['b'])))
    z = _silu(_bn(_mm(z, params['ang2']['W'], params['ang2']['b'])))

    egc_t = _make_egc_sc(T, E)
    egc_e = _make_egc_sc(E, _N_PAD)
    for lp in params['alignn']:
        m, z = _egc(y, z, i_s_t, jp_t, rs_t, lp['edge'], E, E, egc_t)
        xh, y = _egc(xh, m, ie_s, je_p, rs_e, lp['node'], _N_PAD, N, egc_e)
    for gp in params['gcn']:
        xh, y = _egc(xh, y, ie_s, je_p, rs_e, gp, _N_PAD, N, egc_e)
    sums = jax.ops.segment_sum(xh, batch, num_segments=NG)
    cnt = jax.ops.segment_sum(jnp.ones((N, 1), jnp.float32), batch, num_segments=NG)
    h = sums / jnp.maximum(cnt, 1.0)
    return h @ params['out']['W'] + params['out']['b']


# speedup vs baseline: 1.1567x; 1.1567x over previous
import functools

import jax
import jax.numpy as jnp
from jax import lax
from jax.experimental import pallas as pl
from jax.experimental.pallas import tpu as pltpu
from jax.experimental.pallas import tpu_sc as plsc

N = 10000
E = 160000
T = 320000
H = 256
CENTERS = 80
TRIP = 40
NG = 64

_NC = 2
_NS = 16
_NW = _NC * _NS
_B = 40


def _silu(x):
    return x * jax.nn.sigmoid(x)


def _bn(x):
    m = jnp.mean(x, axis=0)
    v = jnp.var(x, axis=0)
    return (x - m) / jnp.sqrt(v + 1e-5)


def _rbf(d, vmin, vmax, bins):
    centers = jnp.linspace(vmin, vmax, bins)
    gamma = 1.0 / ((vmax - vmin) / (bins - 1))
    return jnp.exp(-gamma * (d - centers) ** 2)


def _mm_body(x_ref, w_ref, b_ref, o_ref):
    o_ref[...] = (
        jnp.dot(x_ref[...], w_ref[...], preferred_element_type=jnp.float32)
        + b_ref[...]
    )


def _mm(x, W, b, bm=1000):
    R, K = x.shape
    O = W.shape[1]
    return pl.pallas_call(
        _mm_body,
        grid=(R // bm,),
        in_specs=[
            pl.BlockSpec((bm, K), lambda r: (r, 0)),
            pl.BlockSpec((K, O), lambda r: (0, 0)),
            pl.BlockSpec((1, O), lambda r: (0, 0)),
        ],
        out_specs=pl.BlockSpec((bm, O), lambda r: (r, 0)),
        out_shape=jax.ShapeDtypeStruct((R, O), jnp.float32),
    )(x, W, b.reshape(1, -1))


_CT = 64


@functools.lru_cache(maxsize=None)
def _make_gate(nt):
    per_w = nt // _NW
    B = 80 if per_w % 80 == 0 else 40
    nblk = per_w // B
    assert nblk * B == per_w

    mesh = plsc.VectorSubcoreMesh(core_axis_name="c", subcore_axis_name="s")

    @functools.partial(
        pl.kernel,
        mesh=mesh,
        out_type=[
            jax.ShapeDtypeStruct((nt, H), jnp.float32),
            jax.ShapeDtypeStruct((nt, H), jnp.float32),
            jax.ShapeDtypeStruct((nt, H), jnp.float32),
        ],
        scratch_types=[
            pltpu.VMEM((B,), jnp.int32),
            pltpu.VMEM((B,), jnp.int32),
            pltpu.VMEM((B, H), jnp.float32),
            pltpu.VMEM((B, H), jnp.float32),
            pltpu.VMEM((B, H), jnp.float32),
            pltpu.VMEM((B, H), jnp.float32),
            pltpu.SemaphoreType.DMA,
        ],
    )
    def gate(i_hbm, j_hbm, es_hbm, ed_hbm, bh_hbm, eg_hbm,
             yg_hbm, sg_hbm, m_hbm,
             ii_v, jj_v, es_v, ed_v, bh_v, eg_v, sem):
        w = lax.axis_index("s") * _NC + lax.axis_index("c")
        base0 = w * per_w

        def blk(g, carry):
            base = base0 + g * B
            pltpu.sync_copy(i_hbm.at[pl.ds(base, B)], ii_v)
            pltpu.sync_copy(j_hbm.at[pl.ds(base, B)], jj_v)
            c1 = pltpu.async_copy(es_hbm.at[ii_v], es_v, sem)
            c2 = pltpu.async_copy(ed_hbm.at[jj_v], ed_v, sem)
            c3 = pltpu.async_copy(bh_hbm.at[jj_v], bh_v, sem)
            c4 = pltpu.async_copy(eg_hbm.at[pl.ds(base, B)], eg_v, sem)
            c1.wait()
            c2.wait()
            c3.wait()
            c4.wait()

            def row(r, cr):
                for cc in range(H // 16):
                    sl = pl.ds(cc * 16, 16)
                    yg = es_v[r, sl] + ed_v[r, sl] + eg_v[r, sl]
                    sig = 1.0 / (1.0 + jnp.exp(-yg))
                    m = bh_v[r, sl] * sig
                    es_v[r, sl] = yg
                    ed_v[r, sl] = sig
                    bh_v[r, sl] = m
                return cr

            lax.fori_loop(0, B, row, 0, unroll=False)
            pltpu.sync_copy(es_v, yg_hbm.at[pl.ds(base, B)])
            pltpu.sync_copy(ed_v, sg_hbm.at[pl.ds(base, B)])
            pltpu.sync_copy(bh_v, m_hbm.at[pl.ds(base, B)])
            return carry

        lax.fori_loop(0, nblk, blk, 0, unroll=False)

    return gate


@functools.lru_cache(maxsize=None)
def _make_segsum(nt, nseg):
    Ct = _CT
    nchunk = nseg // Ct
    assert nchunk * Ct == nseg

    mesh = plsc.VectorSubcoreMesh(core_axis_name="c", subcore_axis_name="s")

    @functools.partial(
        pl.kernel,
        mesh=mesh,
        compiler_params=pltpu.CompilerParams(needs_layout_passes=False),
        out_type=[
            jax.ShapeDtypeStruct((nseg, H), jnp.float32),
            jax.ShapeDtypeStruct((nseg, H), jnp.float32),
        ],
        scratch_types=[
            pltpu.VMEM((nchunk + 17,), jnp.int32),
            pltpu.VMEM((64,), jnp.int32),
            pltpu.VMEM((64, H), jnp.float32),
            pltpu.VMEM((64, H), jnp.float32),
            pltpu.VMEM((Ct + 1, H), jnp.float32),
            pltpu.VMEM((Ct + 1, H), jnp.float32),
            pltpu.SemaphoreType.DMA,
        ],
    )
    def segsum(is_hbm, rs_hbm, m_hbm, sg_hbm, ssh_hbm, ss_hbm,
               rs_v, ivb, mrow, srow, acc_m, acc_s, sem):
        w = lax.axis_index("s") * _NC + lax.axis_index("c")
        pltpu.sync_copy(rs_hbm, rs_v)

        def zrow(r, cr):
            for cc in range(H // 16):
                sl0 = pl.ds(cc * 16, 16)
                acc_m[r, sl0] = jnp.zeros((16,), jnp.float32)
                acc_s[r, sl0] = jnp.zeros((16,), jnp.float32)
            return cr

        lax.fori_loop(0, Ct + 1, zrow, 0, unroll=False)

        cntw = (nchunk - w + _NW - 1) // _NW
        iota16 = lax.broadcasted_iota(jnp.int32, (16,), 0)
        col_i = [iota16 + cc * 16 for cc in range(H // 16)]

        def chunk_body(k, carry):
            c = w + _NW * k
            seg_base = c * Ct
            bv = rs_v[pl.ds(c, 16)]
            start = bv[0]
            end = bv[1]
            ga = (start // 8) * 8
            ngr = jnp.maximum((end - ga + 63) // 64, 0)

            def gbody(g, cr2):
                bp = ga + g * 64
                pltpu.sync_copy(is_hbm.at[pl.ds(bp, 64)], ivb)
                c1 = pltpu.async_copy(m_hbm.at[pl.ds(bp, 64)], mrow, sem)
                c2 = pltpu.async_copy(sg_hbm.at[pl.ds(bp, 64)], srow, sem)
                c1.wait()
                c2.wait()
                for q in range(4):
                    sl = pl.ds(q * 16, 16)
                    iv = ivb[sl]
                    pvec = iota16 + (bp + q * 16)
                    valid = (pvec >= start) & (pvec < end)
                    lv = jnp.where(valid, iv - seg_base, Ct)
                    for rr in range(16):
                        rowi = jnp.zeros((16,), jnp.int32) + lv[rr]
                        for cc in range(H // 16):
                            slc = pl.ds(cc * 16, 16)
                            plsc.addupdate_scatter(
                                acc_m, [rowi, col_i[cc]], mrow[q * 16 + rr, slc])
                            plsc.addupdate_scatter(
                                acc_s, [rowi, col_i[cc]], srow[q * 16 + rr, slc])
                return cr2

            lax.fori_loop(0, ngr, gbody, 0, unroll=False)
            pltpu.sync_copy(acc_m.at[pl.ds(0, Ct)], ssh_hbm.at[pl.ds(seg_base, Ct)])
            pltpu.sync_copy(acc_s.at[pl.ds(0, Ct)], ss_hbm.at[pl.ds(seg_base, Ct)])
            lax.fori_loop(0, Ct, zrow, 0, unroll=False)
            return carry

        lax.fori_loop(0, cntw, chunk_body, 0, unroll=False)

    return segsum


_N_PAD = 10240


def _prep(seg_ids, other_ids, nseg_pad):
    perm = jnp.argsort(seg_ids).astype(jnp.int32)
    i_s = seg_ids[perm].astype(jnp.int32)
    jp = other_ids[perm].astype(jnp.int32)
    nchunk = nseg_pad // _CT
    bounds = (jnp.arange(nchunk + 1, dtype=jnp.int32) * _CT)
    rs = jnp.searchsorted(i_s, bounds).astype(jnp.int32)
    rs = jnp.concatenate([rs, jnp.zeros((16,), jnp.int32)])
    pad = jnp.zeros((128,), jnp.int32)
    return perm, jnp.concatenate([i_s, pad]), jnp.concatenate([jp, pad]), rs


def _egc(node, edge_s, i_s, jp, rs, p, nseg_pad, n_seg, gate, segsum):
    es = _mm(node, p['sgW'], p['sgb'])
    ed = _mm(node, p['dgW'], p['dgb'])
    bh = _mm(node, p['duW'], p['dub'])
    su = _mm(node, p['suW'], p['sub'])
    eg = _mm(edge_s, p['egW'], p['egb'])
    yg, sg, m = gate(i_s, jp, es, ed, bh, eg)
    ssh, ss = segsum(i_s, rs, m, sg)
    h = ssh[:n_seg] / (ss[:n_seg] + 1e-6)
    xq = _silu(_bn(su + h))
    yq = _silu(_bn(yg))
    return node + xq, edge_s + yq


def kernel(x, edge_index, edge_index_triplets, dist, angle, batch, params):
    ie = edge_index[0]
    je = edge_index[1]
    it = edge_index_triplets[0]
    jt = edge_index_triplets[1]

    perm_e, ie_s, je_p, rs_e = _prep(ie, je, _N_PAD)
    rank_e = jnp.zeros((E,), jnp.int32).at[perm_e].set(
        jnp.arange(E, dtype=jnp.int32), unique_indices=True)
    itp = rank_e[it]
    jtp = rank_e[jt]
    perm_t, i_s_t, jp_t, rs_t = _prep(itp, jtp, E)

    xh = _silu(_bn(x @ params['atom']['W'] + params['atom']['b']))
    y = _rbf(dist[perm_e], 0.0, 8.0, CENTERS)
    y = _silu(_bn(_mm(y, params['edge1']['W'], params['edge1']['b'])))
    y = _silu(_bn(_mm(y, params['edge2']['W'], params['edge2']['b'])))
    z = _rbf(angle[perm_t], -1.0, 1.0, TRIP)
    z = _silu(_bn(_mm(z, params['ang1']['W'], params['ang1']['b'])))
    z = _silu(_bn(_mm(z, params['ang2']['W'], params['ang2']['b'])))

    gate_t = _make_gate(T)
    gate_e = _make_gate(E)
    seg_t = _make_segsum(T, E)
    seg_e = _make_segsum(E, _N_PAD)
    for lp in params['alignn']:
        m, z = _egc(y, z, i_s_t, jp_t, rs_t, lp['edge'], E, E, gate_t, seg_t)
        xh, y = _egc(xh, m, ie_s, je_p, rs_e, lp['node'], _N_PAD, N, gate_e, seg_e)
    for gp in params['gcn']:
        xh, y = _egc(xh, y, ie_s, je_p, rs_e, gp, _N_PAD, N, gate_e, seg_e)
    sums = jax.ops.segment_sum(xh, batch, num_segments=NG)
    cnt = jax.ops.segment_sum(jnp.ones((N, 1), jnp.float32), batch, num_segments=NG)
    h = sums / jnp.maximum(cnt, 1.0)
    return h @ params['out']['W'] + params['out']['b']


# speedup vs baseline: 1.4368x; 1.2422x over previous
import functools

import jax
import jax.numpy as jnp
from jax import lax
from jax.experimental import pallas as pl
from jax.experimental.pallas import tpu as pltpu
from jax.experimental.pallas import tpu_sc as plsc

N = 10000
E = 160000
T = 320000
H = 256
CENTERS = 80
TRIP = 40
NG = 64

_NC = 2
_NS = 16
_NW = _NC * _NS
_B = 40


def _silu(x):
    return x * jax.nn.sigmoid(x)


def _bn(x):
    m = jnp.mean(x, axis=0)
    v = jnp.var(x, axis=0)
    return (x - m) / jnp.sqrt(v + 1e-5)


def _rbf(d, vmin, vmax, bins):
    centers = jnp.linspace(vmin, vmax, bins)
    gamma = 1.0 / ((vmax - vmin) / (bins - 1))
    return jnp.exp(-gamma * (d - centers) ** 2)


def _mm_body(x_ref, w_ref, b_ref, o_ref):
    o_ref[...] = (
        jnp.dot(x_ref[...], w_ref[...], preferred_element_type=jnp.float32)
        + b_ref[...]
    )


def _mm(x, W, b, bm=1000):
    R, K = x.shape
    O = W.shape[1]
    return pl.pallas_call(
        _mm_body,
        grid=(R // bm,),
        in_specs=[
            pl.BlockSpec((bm, K), lambda r: (r, 0)),
            pl.BlockSpec((K, O), lambda r: (0, 0)),
            pl.BlockSpec((1, O), lambda r: (0, 0)),
        ],
        out_specs=pl.BlockSpec((bm, O), lambda r: (r, 0)),
        out_shape=jax.ShapeDtypeStruct((R, O), jnp.float32),
    )(x, W, b.reshape(1, -1))


_CT = 64


@functools.lru_cache(maxsize=None)
def _make_gate(nt):
    per_w = nt // _NW
    B = 80 if per_w % 80 == 0 else 40
    nblk = per_w // B
    assert nblk * B == per_w

    mesh = plsc.VectorSubcoreMesh(core_axis_name="c", subcore_axis_name="s")

    @functools.partial(
        pl.kernel,
        mesh=mesh,
        out_type=[
            jax.ShapeDtypeStruct((nt, H), jnp.float32),
            jax.ShapeDtypeStruct((nt, H), jnp.float32),
            jax.ShapeDtypeStruct((nt, H), jnp.float32),
        ],
        scratch_types=[
            pltpu.VMEM((B,), jnp.int32),
            pltpu.VMEM((B,), jnp.int32),
            pltpu.VMEM((B, H), jnp.float32),
            pltpu.VMEM((B, H), jnp.float32),
            pltpu.VMEM((B, H), jnp.float32),
            pltpu.VMEM((B, H), jnp.float32),
            pltpu.SemaphoreType.DMA,
        ],
    )
    def gate(i_hbm, j_hbm, es_hbm, ed_hbm, bh_hbm, eg_hbm,
             yg_hbm, sg_hbm, m_hbm,
             ii_v, jj_v, es_v, ed_v, bh_v, eg_v, sem):
        w = lax.axis_index("s") * _NC + lax.axis_index("c")
        base0 = w * per_w

        def blk(g, carry):
            base = base0 + g * B
            pltpu.sync_copy(i_hbm.at[pl.ds(base, B)], ii_v)
            pltpu.sync_copy(j_hbm.at[pl.ds(base, B)], jj_v)
            c1 = pltpu.async_copy(es_hbm.at[ii_v], es_v, sem)
            c2 = pltpu.async_copy(ed_hbm.at[jj_v], ed_v, sem)
            c3 = pltpu.async_copy(bh_hbm.at[jj_v], bh_v, sem)
            c4 = pltpu.async_copy(eg_hbm.at[pl.ds(base, B)], eg_v, sem)
            c1.wait()
            c2.wait()
            c3.wait()
            c4.wait()

            def row(r, cr):
                for cc in range(H // 16):
                    sl = pl.ds(cc * 16, 16)
                    yg = es_v[r, sl] + ed_v[r, sl] + eg_v[r, sl]
                    sig = 1.0 / (1.0 + jnp.exp(-yg))
                    m = bh_v[r, sl] * sig
                    es_v[r, sl] = yg
                    ed_v[r, sl] = sig
                    bh_v[r, sl] = m
                return cr

            lax.fori_loop(0, B, row, 0, unroll=False)
            pltpu.sync_copy(es_v, yg_hbm.at[pl.ds(base, B)])
            pltpu.sync_copy(ed_v, sg_hbm.at[pl.ds(base, B)])
            pltpu.sync_copy(bh_v, m_hbm.at[pl.ds(base, B)])
            return carry

        lax.fori_loop(0, nblk, blk, 0, unroll=False)

    return gate


@functools.lru_cache(maxsize=None)
def _make_segsum(nt, nseg, use_perm):
    Ct = _CT
    nchunk = nseg // Ct
    assert nchunk * Ct == nseg

    mesh = plsc.VectorSubcoreMesh(core_axis_name="c", subcore_axis_name="s")

    @functools.partial(
        pl.kernel,
        mesh=mesh,
        compiler_params=pltpu.CompilerParams(needs_layout_passes=False),
        out_type=[
            jax.ShapeDtypeStruct((nseg, H), jnp.float32),
            jax.ShapeDtypeStruct((nseg, H), jnp.float32),
        ],
        scratch_types=[
            pltpu.VMEM((nchunk + 17,), jnp.int32),
            pltpu.VMEM((64,), jnp.int32),
            pltpu.VMEM((64,), jnp.int32),
            pltpu.VMEM((64, H), jnp.float32),
            pltpu.VMEM((64, H), jnp.float32),
            pltpu.VMEM((Ct + 1, H), jnp.float32),
            pltpu.VMEM((Ct + 1, H), jnp.float32),
            pltpu.SemaphoreType.DMA,
        ],
    )
    def segsum(is_hbm, perm_hbm, rs_hbm, m_hbm, sg_hbm, ssh_hbm, ss_hbm,
               rs_v, ivb, posG, mrow, srow, acc_m, acc_s, sem):
        w = lax.axis_index("s") * _NC + lax.axis_index("c")
        pltpu.sync_copy(rs_hbm, rs_v)

        def zrow(r, cr):
            for cc in range(H // 16):
                sl0 = pl.ds(cc * 16, 16)
                acc_m[r, sl0] = jnp.zeros((16,), jnp.float32)
                acc_s[r, sl0] = jnp.zeros((16,), jnp.float32)
            return cr

        lax.fori_loop(0, Ct + 1, zrow, 0, unroll=False)

        cntw = (nchunk - w + _NW - 1) // _NW
        iota16 = lax.broadcasted_iota(jnp.int32, (16,), 0)
        col_i = [iota16 + cc * 16 for cc in range(H // 16)]

        def chunk_body(k, carry):
            c = w + _NW * k
            seg_base = c * Ct
            bv = rs_v[pl.ds(c, 16)]
            start = bv[0]
            end = bv[1]
            ga = (start // 8) * 8
            ngr = jnp.maximum((end - ga + 63) // 64, 0)

            def gbody(g, cr2):
                bp = ga + g * 64
                pltpu.sync_copy(is_hbm.at[pl.ds(bp, 64)], ivb)
                if use_perm:
                    pltpu.sync_copy(perm_hbm.at[pl.ds(bp, 64)], posG)
                    c1 = pltpu.async_copy(m_hbm.at[posG], mrow, sem)
                    c2 = pltpu.async_copy(sg_hbm.at[posG], srow, sem)
                else:
                    c1 = pltpu.async_copy(m_hbm.at[pl.ds(bp, 64)], mrow, sem)
                    c2 = pltpu.async_copy(sg_hbm.at[pl.ds(bp, 64)], srow, sem)
                c1.wait()
                c2.wait()
                for q in range(4):
                    sl = pl.ds(q * 16, 16)
                    iv = ivb[sl]
                    pvec = iota16 + (bp + q * 16)
                    valid = (pvec >= start) & (pvec < end)
                    lv = jnp.where(valid, iv - seg_base, Ct)
                    for rr in range(16):
                        rowi = jnp.zeros((16,), jnp.int32) + lv[rr]
                        for cc in range(H // 16):
                            slc = pl.ds(cc * 16, 16)
                            plsc.addupdate_scatter(
                                acc_m, [rowi, col_i[cc]], mrow[q * 16 + rr, slc])
                            plsc.addupdate_scatter(
                                acc_s, [rowi, col_i[cc]], srow[q * 16 + rr, slc])
                return cr2

            lax.fori_loop(0, ngr, gbody, 0, unroll=False)
            pltpu.sync_copy(acc_m.at[pl.ds(0, Ct)], ssh_hbm.at[pl.ds(seg_base, Ct)])
            pltpu.sync_copy(acc_s.at[pl.ds(0, Ct)], ss_hbm.at[pl.ds(seg_base, Ct)])
            lax.fori_loop(0, Ct, zrow, 0, unroll=False)
            return carry

        lax.fori_loop(0, cntw, chunk_body, 0, unroll=False)

    return segsum


_N_PAD = 10240


def _prep(seg_ids, other_ids, nseg_pad):
    perm = jnp.argsort(seg_ids).astype(jnp.int32)
    i_s = seg_ids[perm].astype(jnp.int32)
    jp = other_ids[perm].astype(jnp.int32)
    nchunk = nseg_pad // _CT
    bounds = (jnp.arange(nchunk + 1, dtype=jnp.int32) * _CT)
    rs = jnp.searchsorted(i_s, bounds).astype(jnp.int32)
    rs = jnp.concatenate([rs, jnp.zeros((16,), jnp.int32)])
    pad = jnp.zeros((128,), jnp.int32)
    return (jnp.concatenate([perm, pad]), jnp.concatenate([i_s, pad]),
            jnp.concatenate([jp, pad]), rs)


def _egc(node, edge_f, gi, gj, i_s, perm, rs, p, n_seg, gate, segsum):
    es = _mm(node, p['sgW'], p['sgb'])
    ed = _mm(node, p['dgW'], p['dgb'])
    bh = _mm(node, p['duW'], p['dub'])
    su = _mm(node, p['suW'], p['sub'])
    eg = _mm(edge_f, p['egW'], p['egb'])
    yg, sg, m = gate(gi, gj, es, ed, bh, eg)
    ssh, ss = segsum(i_s, perm, rs, m, sg)
    h = ssh[:n_seg] / (ss[:n_seg] + 1e-6)
    xq = _silu(_bn(su + h))
    yq = _silu(_bn(yg))
    return node + xq, edge_f + yq


def kernel(x, edge_index, edge_index_triplets, dist, angle, batch, params):
    ie = edge_index[0]
    je = edge_index[1]
    it = edge_index_triplets[0]
    jt = edge_index_triplets[1]

    perm_t, i_s_t, jp_t, rs_t = _prep(it, jt, E)
    perm_e, ie_s, _je_s, rs_e = _prep(ie, je, _N_PAD)

    xh = _silu(_bn(x @ params['atom']['W'] + params['atom']['b']))
    y = _rbf(dist, 0.0, 8.0, CENTERS)
    y = _silu(_bn(_mm(y, params['edge1']['W'], params['edge1']['b'])))
    y = _silu(_bn(_mm(y, params['edge2']['W'], params['edge2']['b'])))
    z = _rbf(angle[perm_t[:T]], -1.0, 1.0, TRIP)
    z = _silu(_bn(_mm(z, params['ang1']['W'], params['ang1']['b'])))
    z = _silu(_bn(_mm(z, params['ang2']['W'], params['ang2']['b'])))

    gate_t = _make_gate(T)
    gate_e = _make_gate(E)
    seg_t = _make_segsum(T, E, False)
    seg_e = _make_segsum(E, _N_PAD, True)
    ie32 = ie.astype(jnp.int32)
    je32 = je.astype(jnp.int32)
    for lp in params['alignn']:
        m, z = _egc(y, z, i_s_t, jp_t, i_s_t, i_s_t, rs_t, lp['edge'], E,
                    gate_t, seg_t)
        xh, y = _egc(xh, m, ie32, je32, ie_s, perm_e, rs_e, lp['node'], N,
                     gate_e, seg_e)
    for gp in params['gcn']:
        xh, y = _egc(xh, y, ie32, je32, ie_s, perm_e, rs_e, gp, N,
                     gate_e, seg_e)
    sums = jax.ops.segment_sum(xh, batch, num_segments=NG)
    cnt = jax.ops.segment_sum(jnp.ones((N, 1), jnp.float32), batch, num_segments=NG)
    h = sums / jnp.maximum(cnt, 1.0)
    return h @ params['out']['W'] + params['out']['b']


# speedup vs baseline: 1.4532x; 1.0114x over previous
import functools

import jax
import jax.numpy as jnp
from jax import lax
from jax.experimental import pallas as pl
from jax.experimental.pallas import tpu as pltpu
from jax.experimental.pallas import tpu_sc as plsc

N = 10000
E = 160000
T = 320000
H = 256
CENTERS = 80
TRIP = 40
NG = 64

_NC = 2
_NS = 16
_NW = _NC * _NS
_B = 40


def _silu(x):
    return x * jax.nn.sigmoid(x)


def _bn(x):
    m = jnp.mean(x, axis=0)
    v = jnp.var(x, axis=0)
    return (x - m) / jnp.sqrt(v + 1e-5)


def _rbf(d, vmin, vmax, bins):
    centers = jnp.linspace(vmin, vmax, bins)
    gamma = 1.0 / ((vmax - vmin) / (bins - 1))
    return jnp.exp(-gamma * (d - centers) ** 2)


def _mm_body(x_ref, w_ref, b_ref, o_ref):
    o_ref[...] = (
        jnp.dot(x_ref[...], w_ref[...], preferred_element_type=jnp.float32)
        + b_ref[...]
    )


def _mm(x, W, b, bm=1000):
    R, K = x.shape
    O = W.shape[1]
    return pl.pallas_call(
        _mm_body,
        grid=(R // bm,),
        in_specs=[
            pl.BlockSpec((bm, K), lambda r: (r, 0)),
            pl.BlockSpec((K, O), lambda r: (0, 0)),
            pl.BlockSpec((1, O), lambda r: (0, 0)),
        ],
        out_specs=pl.BlockSpec((bm, O), lambda r: (r, 0)),
        out_shape=jax.ShapeDtypeStruct((R, O), jnp.float32),
    )(x, W, b.reshape(1, -1))


_CT = 128


@functools.lru_cache(maxsize=None)
def _make_gate(nt):
    per_w = nt // _NW
    B = 80 if per_w % 80 == 0 else 40
    nblk = per_w // B
    assert nblk * B == per_w

    mesh = plsc.VectorSubcoreMesh(core_axis_name="c", subcore_axis_name="s")

    @functools.partial(
        pl.kernel,
        mesh=mesh,
        out_type=[
            jax.ShapeDtypeStruct((nt, H), jnp.float32),
            jax.ShapeDtypeStruct((nt, H), jnp.float32),
            jax.ShapeDtypeStruct((nt, H), jnp.float32),
        ],
        scratch_types=[
            pltpu.VMEM((B,), jnp.int32),
            pltpu.VMEM((B,), jnp.int32),
            pltpu.VMEM((B, H), jnp.float32),
            pltpu.VMEM((B, H), jnp.float32),
            pltpu.VMEM((B, H), jnp.float32),
            pltpu.VMEM((B, H), jnp.float32),
            pltpu.SemaphoreType.DMA,
        ],
    )
    def gate(i_hbm, j_hbm, es_hbm, ed_hbm, bh_hbm, eg_hbm,
             yg_hbm, sg_hbm, m_hbm,
             ii_v, jj_v, es_v, ed_v, bh_v, eg_v, sem):
        w = lax.axis_index("s") * _NC + lax.axis_index("c")
        base0 = w * per_w

        def blk(g, carry):
            base = base0 + g * B
            pltpu.sync_copy(i_hbm.at[pl.ds(base, B)], ii_v)
            pltpu.sync_copy(j_hbm.at[pl.ds(base, B)], jj_v)
            c1 = pltpu.async_copy(es_hbm.at[ii_v], es_v, sem)
            c2 = pltpu.async_copy(ed_hbm.at[jj_v], ed_v, sem)
            c3 = pltpu.async_copy(bh_hbm.at[jj_v], bh_v, sem)
            c4 = pltpu.async_copy(eg_hbm.at[pl.ds(base, B)], eg_v, sem)
            c1.wait()
            c2.wait()
            c3.wait()
            c4.wait()

            def row(r, cr):
                for cc in range(H // 16):
                    sl = pl.ds(cc * 16, 16)
                    yg = es_v[r, sl] + ed_v[r, sl] + eg_v[r, sl]
                    sig = 1.0 / (1.0 + jnp.exp(-yg))
                    m = bh_v[r, sl] * sig
                    es_v[r, sl] = yg
                    ed_v[r, sl] = sig
                    bh_v[r, sl] = m
                return cr

            lax.fori_loop(0, B, row, 0, unroll=2)
            pltpu.sync_copy(es_v, yg_hbm.at[pl.ds(base, B)])
            pltpu.sync_copy(ed_v, sg_hbm.at[pl.ds(base, B)])
            pltpu.sync_copy(bh_v, m_hbm.at[pl.ds(base, B)])
            return carry

        lax.fori_loop(0, nblk, blk, 0, unroll=False)

    return gate


@functools.lru_cache(maxsize=None)
def _make_segsum(nt, nseg, use_perm):
    Ct = _CT
    nchunk = nseg // Ct
    assert nchunk * Ct == nseg

    mesh = plsc.VectorSubcoreMesh(core_axis_name="c", subcore_axis_name="s")

    @functools.partial(
        pl.kernel,
        mesh=mesh,
        compiler_params=pltpu.CompilerParams(needs_layout_passes=False),
        out_type=[
            jax.ShapeDtypeStruct((nseg, H), jnp.float32),
            jax.ShapeDtypeStruct((nseg, H), jnp.float32),
        ],
        scratch_types=[
            pltpu.VMEM((nchunk + 17,), jnp.int32),
            pltpu.VMEM((64,), jnp.int32),
            pltpu.VMEM((64,), jnp.int32),
            pltpu.VMEM((64, H), jnp.float32),
            pltpu.VMEM((64, H), jnp.float32),
            pltpu.VMEM((Ct + 1, H), jnp.float32),
            pltpu.VMEM((Ct + 1, H), jnp.float32),
            pltpu.SemaphoreType.DMA,
        ],
    )
    def segsum(is_hbm, perm_hbm, rs_hbm, m_hbm, sg_hbm, ssh_hbm, ss_hbm,
               rs_v, ivb, posG, mrow, srow, acc_m, acc_s, sem):
        w = lax.axis_index("s") * _NC + lax.axis_index("c")
        pltpu.sync_copy(rs_hbm, rs_v)

        def zrow(r, cr):
            for cc in range(H // 16):
                sl0 = pl.ds(cc * 16, 16)
                acc_m[r, sl0] = jnp.zeros((16,), jnp.float32)
                acc_s[r, sl0] = jnp.zeros((16,), jnp.float32)
            return cr

        lax.fori_loop(0, Ct + 1, zrow, 0, unroll=False)

        cntw = (nchunk - w + _NW - 1) // _NW
        iota16 = lax.broadcasted_iota(jnp.int32, (16,), 0)
        col_i = [iota16 + cc * 16 for cc in range(H // 16)]

        def chunk_body(k, carry):
            c = w + _NW * k
            seg_base = c * Ct
            bv = rs_v[pl.ds(c, 16)]
            start = bv[0]
            end = bv[1]
            ga = (start // 8) * 8
            ngr = jnp.maximum((end - ga + 63) // 64, 0)

            def gbody(g, cr2):
                bp = ga + g * 64
                pltpu.sync_copy(is_hbm.at[pl.ds(bp, 64)], ivb)
                if use_perm:
                    pltpu.sync_copy(perm_hbm.at[pl.ds(bp, 64)], posG)
                    c1 = pltpu.async_copy(m_hbm.at[posG], mrow, sem)
                    c2 = pltpu.async_copy(sg_hbm.at[posG], srow, sem)
                else:
                    c1 = pltpu.async_copy(m_hbm.at[pl.ds(bp, 64)], mrow, sem)
                    c2 = pltpu.async_copy(sg_hbm.at[pl.ds(bp, 64)], srow, sem)
                c1.wait()
                c2.wait()
                for q in range(4):
                    sl = pl.ds(q * 16, 16)
                    iv = ivb[sl]
                    pvec = iota16 + (bp + q * 16)
                    valid = (pvec >= start) & (pvec < end)
                    lv = jnp.where(valid, iv - seg_base, Ct)
                    for rr in range(16):
                        rowi = jnp.zeros((16,), jnp.int32) + lv[rr]
                        for cc in range(H // 16):
                            slc = pl.ds(cc * 16, 16)
                            plsc.addupdate_scatter(
                                acc_m, [rowi, col_i[cc]], mrow[q * 16 + rr, slc])
                            plsc.addupdate_scatter(
                                acc_s, [rowi, col_i[cc]], srow[q * 16 + rr, slc])
                return cr2

            lax.fori_loop(0, ngr, gbody, 0, unroll=False)
            pltpu.sync_copy(acc_m.at[pl.ds(0, Ct)], ssh_hbm.at[pl.ds(seg_base, Ct)])
            pltpu.sync_copy(acc_s.at[pl.ds(0, Ct)], ss_hbm.at[pl.ds(seg_base, Ct)])
            lax.fori_loop(0, Ct, zrow, 0, unroll=False)
            return carry

        lax.fori_loop(0, cntw, chunk_body, 0, unroll=False)

    return segsum


_N_PAD = 10240


def _prep(seg_ids, other_ids, nseg_pad):
    perm = jnp.argsort(seg_ids).astype(jnp.int32)
    i_s = seg_ids[perm].astype(jnp.int32)
    jp = other_ids[perm].astype(jnp.int32)
    nchunk = nseg_pad // _CT
    bounds = (jnp.arange(nchunk + 1, dtype=jnp.int32) * _CT)
    rs = jnp.searchsorted(i_s, bounds).astype(jnp.int32)
    rs = jnp.concatenate([rs, jnp.zeros((16,), jnp.int32)])
    pad = jnp.zeros((128,), jnp.int32)
    return (jnp.concatenate([perm, pad]), jnp.concatenate([i_s, pad]),
            jnp.concatenate([jp, pad]), rs)


def _egc(node, edge_f, gi, gj, i_s, perm, rs, p, n_seg, gate, segsum):
    es = _mm(node, p['sgW'], p['sgb'])
    ed = _mm(node, p['dgW'], p['dgb'])
    bh = _mm(node, p['duW'], p['dub'])
    su = _mm(node, p['suW'], p['sub'])
    eg = _mm(edge_f, p['egW'], p['egb'])
    yg, sg, m = gate(gi, gj, es, ed, bh, eg)
    ssh, ss = segsum(i_s, perm, rs, m, sg)
    h = ssh[:n_seg] / (ss[:n_seg] + 1e-6)
    xq = _silu(_bn(su + h))
    yq = _silu(_bn(yg))
    return node + xq, edge_f + yq


def kernel(x, edge_index, edge_index_triplets, dist, angle, batch, params):
    ie = edge_index[0]
    je = edge_index[1]
    it = edge_index_triplets[0]
    jt = edge_index_triplets[1]

    perm_t, i_s_t, jp_t, rs_t = _prep(it, jt, E)
    perm_e, ie_s, _je_s, rs_e = _prep(ie, je, _N_PAD)

    xh = _silu(_bn(x @ params['atom']['W'] + params['atom']['b']))
    y = _rbf(dist, 0.0, 8.0, CENTERS)
    y = _silu(_bn(_mm(y, params['edge1']['W'], params['edge1']['b'])))
    y = _silu(_bn(_mm(y, params['edge2']['W'], params['edge2']['b'])))
    z = _rbf(angle[perm_t[:T]], -1.0, 1.0, TRIP)
    z = _silu(_bn(_mm(z, params['ang1']['W'], params['ang1']['b'])))
    z = _silu(_bn(_mm(z, params['ang2']['W'], params['ang2']['b'])))

    gate_t = _make_gate(T)
    gate_e = _make_gate(E)
    seg_t = _make_segsum(T, E, False)
    seg_e = _make_segsum(E, _N_PAD, True)
    ie32 = ie.astype(jnp.int32)
    je32 = je.astype(jnp.int32)
    for lp in params['alignn']:
        m, z = _egc(y, z, i_s_t, jp_t, i_s_t, i_s_t, rs_t, lp['edge'], E,
                    gate_t, seg_t)
        xh, y = _egc(xh, m, ie32, je32, ie_s, perm_e, rs_e, lp['node'], N,
                     gate_e, seg_e)
    for gp in params['gcn']:
        xh, y = _egc(xh, y, ie32, je32, ie_s, perm_e, rs_e, gp, N,
                     gate_e, seg_e)
    sums = jax.ops.segment_sum(xh, batch, num_segments=NG)
    cnt = jax.ops.segment_sum(jnp.ones((N, 1), jnp.float32), batch, num_segments=NG)
    h = sums / jnp.maximum(cnt, 1.0)
    return h @ params['out']['W'] + params['out']['b']


# speedup vs baseline: 1.4745x; 1.0147x over previous
import functools

import jax
import jax.numpy as jnp
from jax import lax
from jax.experimental import pallas as pl
from jax.experimental.pallas import tpu as pltpu
from jax.experimental.pallas import tpu_sc as plsc

N = 10000
E = 160000
T = 320000
H = 256
CENTERS = 80
TRIP = 40
NG = 64

_NC = 2
_NS = 16
_NW = _NC * _NS
_B = 40


def _silu(x):
    return x * jax.nn.sigmoid(x)


def _bn(x):
    m = jnp.mean(x, axis=0)
    v = jnp.var(x, axis=0)
    return (x - m) / jnp.sqrt(v + 1e-5)


def _rbf(d, vmin, vmax, bins):
    centers = jnp.linspace(vmin, vmax, bins)
    gamma = 1.0 / ((vmax - vmin) / (bins - 1))
    return jnp.exp(-gamma * (d - centers) ** 2)


def _mm_body(x_ref, w_ref, b_ref, o_ref):
    o_ref[...] = (
        jnp.dot(x_ref[...], w_ref[...], preferred_element_type=jnp.float32)
        + b_ref[...]
    )


def _mm(x, W, b, bm=1000):
    R, K = x.shape
    O = W.shape[1]
    return pl.pallas_call(
        _mm_body,
        grid=(R // bm,),
        in_specs=[
            pl.BlockSpec((bm, K), lambda r: (r, 0)),
            pl.BlockSpec((K, O), lambda r: (0, 0)),
            pl.BlockSpec((1, O), lambda r: (0, 0)),
        ],
        out_specs=pl.BlockSpec((bm, O), lambda r: (r, 0)),
        out_shape=jax.ShapeDtypeStruct((R, O), jnp.float32),
    )(x, W, b.reshape(1, -1))


_CT = 128


@functools.lru_cache(maxsize=None)
def _make_gate(nt):
    per_w = nt // _NW
    B = 80 if per_w % 80 == 0 else 40
    nblk = per_w // B
    assert nblk * B == per_w

    mesh = plsc.VectorSubcoreMesh(core_axis_name="c", subcore_axis_name="s")

    @functools.partial(
        pl.kernel,
        mesh=mesh,
        out_type=[
            jax.ShapeDtypeStruct((nt, H), jnp.float32),
            jax.ShapeDtypeStruct((nt, H), jnp.float32),
            jax.ShapeDtypeStruct((nt, H), jnp.float32),
        ],
        scratch_types=[
            pltpu.VMEM((B,), jnp.int32),
            pltpu.VMEM((B,), jnp.int32),
            pltpu.VMEM((B, H), jnp.float32),
            pltpu.VMEM((B, H), jnp.float32),
            pltpu.VMEM((B, H), jnp.float32),
            pltpu.VMEM((B, H), jnp.float32),
            pltpu.SemaphoreType.DMA,
        ],
    )
    def gate(i_hbm, j_hbm, es_hbm, ed_hbm, bh_hbm, eg_hbm,
             yg_hbm, sg_hbm, m_hbm,
             ii_v, jj_v, es_v, ed_v, bh_v, eg_v, sem):
        w = lax.axis_index("s") * _NC + lax.axis_index("c")
        base0 = w * per_w

        def blk(g, carry):
            base = base0 + g * B
            pltpu.sync_copy(i_hbm.at[pl.ds(base, B)], ii_v)
            pltpu.sync_copy(j_hbm.at[pl.ds(base, B)], jj_v)
            c1 = pltpu.async_copy(es_hbm.at[ii_v], es_v, sem)
            c2 = pltpu.async_copy(ed_hbm.at[jj_v], ed_v, sem)
            c3 = pltpu.async_copy(bh_hbm.at[jj_v], bh_v, sem)
            c4 = pltpu.async_copy(eg_hbm.at[pl.ds(base, B)], eg_v, sem)
            c1.wait()
            c2.wait()
            c3.wait()
            c4.wait()

            def row(r, cr):
                for cc in range(H // 16):
                    sl = pl.ds(cc * 16, 16)
                    yg = es_v[r, sl] + ed_v[r, sl] + eg_v[r, sl]
                    sig = 1.0 / (1.0 + jnp.exp(-yg))
                    m = bh_v[r, sl] * sig
                    es_v[r, sl] = yg
                    ed_v[r, sl] = sig
                    bh_v[r, sl] = m
                return cr

            lax.fori_loop(0, B, row, 0, unroll=2)
            pltpu.sync_copy(es_v, yg_hbm.at[pl.ds(base, B)])
            pltpu.sync_copy(ed_v, sg_hbm.at[pl.ds(base, B)])
            pltpu.sync_copy(bh_v, m_hbm.at[pl.ds(base, B)])
            return carry

        lax.fori_loop(0, nblk, blk, 0, unroll=False)

    return gate


@functools.lru_cache(maxsize=None)
def _make_segsum(nt, nseg, use_perm):
    Ct = _CT
    nchunk = nseg // Ct
    assert nchunk * Ct == nseg

    mesh = plsc.VectorSubcoreMesh(core_axis_name="c", subcore_axis_name="s")

    @functools.partial(
        pl.kernel,
        mesh=mesh,
        compiler_params=pltpu.CompilerParams(needs_layout_passes=False),
        out_type=[
            jax.ShapeDtypeStruct((nseg, H), jnp.float32),
            jax.ShapeDtypeStruct((nseg, H), jnp.float32),
        ],
        scratch_types=[
            pltpu.VMEM((nchunk + 17,), jnp.int32),
            pltpu.VMEM((64,), jnp.int32),
            pltpu.VMEM((64,), jnp.int32),
            pltpu.VMEM((64, H), jnp.float32),
            pltpu.VMEM((64, H), jnp.float32),
            pltpu.VMEM((Ct + 1, H), jnp.float32),
            pltpu.VMEM((Ct + 1, H), jnp.float32),
            pltpu.SemaphoreType.DMA,
        ],
    )
    def segsum(is_hbm, perm_hbm, rs_hbm, m_hbm, sg_hbm, ssh_hbm, ss_hbm,
               rs_v, ivb, posG, mrow, srow, acc_m, acc_s, sem):
        w = lax.axis_index("s") * _NC + lax.axis_index("c")
        pltpu.sync_copy(rs_hbm, rs_v)

        def zrow(r, cr):
            for cc in range(H // 16):
                sl0 = pl.ds(cc * 16, 16)
                acc_m[r, sl0] = jnp.zeros((16,), jnp.float32)
                acc_s[r, sl0] = jnp.zeros((16,), jnp.float32)
            return cr

        lax.fori_loop(0, Ct + 1, zrow, 0, unroll=False)

        cntw = (nchunk - w + _NW - 1) // _NW
        iota16 = lax.broadcasted_iota(jnp.int32, (16,), 0)
        col_i = [iota16 + cc * 16 for cc in range(H // 16)]

        def chunk_body(k, carry):
            c = w + _NW * k
            seg_base = c * Ct
            bv = rs_v[pl.ds(c, 16)]
            start = bv[0]
            end = bv[1]
            ga = (start // 8) * 8
            ngr = jnp.maximum((end - ga + 63) // 64, 0)

            def gbody(g, cr2):
                bp = ga + g * 64
                c0 = pltpu.async_copy(is_hbm.at[pl.ds(bp, 64)], ivb, sem)
                if use_perm:
                    pltpu.sync_copy(perm_hbm.at[pl.ds(bp, 64)], posG)
                    c1 = pltpu.async_copy(m_hbm.at[posG], mrow, sem)
                    c2 = pltpu.async_copy(sg_hbm.at[posG], srow, sem)
                else:
                    c1 = pltpu.async_copy(m_hbm.at[pl.ds(bp, 64)], mrow, sem)
                    c2 = pltpu.async_copy(sg_hbm.at[pl.ds(bp, 64)], srow, sem)
                c0.wait()
                c1.wait()
                c2.wait()
                for q in range(4):
                    sl = pl.ds(q * 16, 16)
                    iv = ivb[sl]
                    pvec = iota16 + (bp + q * 16)
                    valid = (pvec >= start) & (pvec < end)
                    lv = jnp.where(valid, iv - seg_base, Ct)
                    for rr in range(16):
                        rowi = jnp.zeros((16,), jnp.int32) + lv[rr]
                        for cc in range(H // 16):
                            slc = pl.ds(cc * 16, 16)
                            plsc.addupdate_scatter(
                                acc_m, [rowi, col_i[cc]], mrow[q * 16 + rr, slc])
                            plsc.addupdate_scatter(
                                acc_s, [rowi, col_i[cc]], srow[q * 16 + rr, slc])
                return cr2

            lax.fori_loop(0, ngr, gbody, 0, unroll=False)
            pltpu.sync_copy(acc_m.at[pl.ds(0, Ct)], ssh_hbm.at[pl.ds(seg_base, Ct)])
            pltpu.sync_copy(acc_s.at[pl.ds(0, Ct)], ss_hbm.at[pl.ds(seg_base, Ct)])
            lax.fori_loop(0, Ct, zrow, 0, unroll=False)
            return carry

        lax.fori_loop(0, cntw, chunk_body, 0, unroll=False)

    return segsum


_N_PAD = 10240


def _prep(seg_ids, other_ids, nseg_pad):
    perm = jnp.argsort(seg_ids).astype(jnp.int32)
    i_s = seg_ids[perm].astype(jnp.int32)
    jp = other_ids[perm].astype(jnp.int32)
    nchunk = nseg_pad // _CT
    bounds = (jnp.arange(nchunk + 1, dtype=jnp.int32) * _CT)
    rs = jnp.searchsorted(i_s, bounds).astype(jnp.int32)
    rs = jnp.concatenate([rs, jnp.zeros((16,), jnp.int32)])
    pad = jnp.zeros((128,), jnp.int32)
    return (jnp.concatenate([perm, pad]), jnp.concatenate([i_s, pad]),
            jnp.concatenate([jp, pad]), rs)


def _egc(node, edge_f, gi, gj, i_s, perm, rs, p, n_seg, gate, segsum):
    es = _mm(node, p['sgW'], p['sgb'])
    ed = _mm(node, p['dgW'], p['dgb'])
    bh = _mm(node, p['duW'], p['dub'])
    su = _mm(node, p['suW'], p['sub'])
    eg = _mm(edge_f, p['egW'], p['egb'])
    yg, sg, m = gate(gi, gj, es, ed, bh, eg)
    ssh, ss = segsum(i_s, perm, rs, m, sg)
    h = ssh[:n_seg] / (ss[:n_seg] + 1e-6)
    xq = _silu(_bn(su + h))
    yq = _silu(_bn(yg))
    return node + xq, edge_f + yq


def kernel(x, edge_index, edge_index_triplets, dist, angle, batch, params):
    ie = edge_index[0]
    je = edge_index[1]
    it = edge_index_triplets[0]
    jt = edge_index_triplets[1]

    perm_t, i_s_t, jp_t, rs_t = _prep(it, jt, E)
    perm_e, ie_s, _je_s, rs_e = _prep(ie, je, _N_PAD)

    xh = _silu(_bn(x @ params['atom']['W'] + params['atom']['b']))
    y = _rbf(dist, 0.0, 8.0, CENTERS)
    y = _silu(_bn(_mm(y, params['edge1']['W'], params['edge1']['b'])))
    y = _silu(_bn(_mm(y, params['edge2']['W'], params['edge2']['b'])))
    z = _rbf(angle[perm_t[:T]], -1.0, 1.0, TRIP)
    z = _silu(_bn(_mm(z, params['ang1']['W'], params['ang1']['b'])))
    z = _silu(_bn(_mm(z, params['ang2']['W'], params['ang2']['b'])))

    gate_t = _make_gate(T)
    gate_e = _make_gate(E)
    seg_t = _make_segsum(T, E, False)
    seg_e = _make_segsum(E, _N_PAD, True)
    ie32 = ie.astype(jnp.int32)
    je32 = je.astype(jnp.int32)
    for lp in params['alignn']:
        m, z = _egc(y, z, i_s_t, jp_t, i_s_t, i_s_t, rs_t, lp['edge'], E,
                    gate_t, seg_t)
        xh, y = _egc(xh, m, ie32, je32, ie_s, perm_e, rs_e, lp['node'], N,
                     gate_e, seg_e)
    for gp in params['gcn']:
        xh, y = _egc(xh, y, ie32, je32, ie_s, perm_e, rs_e, gp, N,
                     gate_e, seg_e)
    sums = jax.ops.segment_sum(xh, batch, num_segments=NG)
    cnt = jax.ops.segment_sum(jnp.ones((N, 1), jnp.float32), batch, num_segments=NG)
    h = sums / jnp.maximum(cnt, 1.0)
    return h @ params['out']['W'] + params['out']['b']
